# baseline (speedup 1.0000x reference)
import functools

import jax
import jax.numpy as jnp
from jax import lax
from jax.experimental import pallas as pl
from jax.experimental.pallas import tpu as pltpu
from jax.experimental.pallas import tpu_sc as plsc

BATCH = 4096
SEQ = 50
EMBED_DIM = 64

NC = 2
NS = 16
NW = NC * NS
B = BATCH * SEQ
B_PER_W = B // NW
CHUNK = 128
NCHUNK = B_PER_W // CHUNK
NBUF = 5
NOUTER = NCHUNK // NBUF


def _gather_body(table_hbm, idx_hbm, out_hbm, idx_v, rows_v, gsem, ssem):
    wid = lax.axis_index("s") * NC + lax.axis_index("c")
    pltpu.sync_copy(idx_hbm.at[wid], idx_v)

    def outer(g, carry):
        for b in range(NBUF):
            @pl.when(g > 0)
            def _wait_store():
                pltpu.make_async_copy(
                    rows_v.at[b], out_hbm.at[wid, 0], ssem.at[b]).wait()
            pltpu.make_async_copy(
                table_hbm.at[idx_v.at[g * NBUF + b]], rows_v.at[b],
                gsem.at[b]).start()
        for b in range(NBUF):
            pltpu.make_async_copy(
                table_hbm.at[idx_v.at[g * NBUF + b]], rows_v.at[b],
                gsem.at[b]).wait()
            pltpu.make_async_copy(
                rows_v.at[b], out_hbm.at[wid, g * NBUF + b],
                ssem.at[b]).start()
        return carry

    lax.fori_loop(0, NOUTER, outer, 0)
    for b in range(NBUF):
        pltpu.make_async_copy(
            rows_v.at[b], out_hbm.at[wid, 0], ssem.at[b]).wait()


@functools.cache
def _make_sc_gather():
    return functools.partial(
        pl.kernel,
        mesh=plsc.VectorSubcoreMesh(
            core_axis_name="c", subcore_axis_name="s",
            num_cores=NC, num_subcores=NS),
        out_type=jax.ShapeDtypeStruct(
            (NW, NCHUNK, CHUNK, EMBED_DIM), jnp.float32),
        scratch_types=[
            pltpu.VMEM((NCHUNK, CHUNK), jnp.int32),
            pltpu.VMEM((NBUF, CHUNK, EMBED_DIM), jnp.float32),
            pltpu.SemaphoreType.DMA((NBUF,)),
            pltpu.SemaphoreType.DMA((NBUF,)),
        ],
        compiler_params=pltpu.CompilerParams(use_tc_tiling_on_sc=False),
    )(_gather_body)


def kernel(input_ids, table):
    ids = input_ids.astype(jnp.int32).reshape(NW, NCHUNK, CHUNK)
    out = _make_sc_gather()(table, ids)
    return out.reshape(BATCH, SEQ, EMBED_DIM)


# speedup vs baseline: 1.0362x; 1.0362x over previous
import functools

import jax
import jax.numpy as jnp
from jax import lax
from jax.experimental import pallas as pl
from jax.experimental.pallas import tpu as pltpu
from jax.experimental.pallas import tpu_sc as plsc

BATCH = 4096
SEQ = 50
EMBED_DIM = 64
PAD_DIM = 128

NC = 2
NS = 16
NW = NC * NS
B = BATCH * SEQ
B_PER_W = B // NW
CHUNK = 128
NCHUNK = B_PER_W // CHUNK
NBUF = 5
NOUTER = NCHUNK // NBUF


def _gather_body(table_hbm, idx_hbm, out_hbm, idx_v, rows_v, gsem, ssem):
    wid = lax.axis_index("s") * NC + lax.axis_index("c")
    pltpu.sync_copy(idx_hbm.at[wid], idx_v)

    def outer(g, carry):
        for b in range(NBUF):
            @pl.when(g > 0)
            def _wait_store():
                pltpu.make_async_copy(
                    rows_v.at[b, :, pl.ds(0, EMBED_DIM)],
                    out_hbm.at[wid, 0], ssem.at[b]).wait()
            pltpu.make_async_copy(
                table_hbm.at[idx_v.at[g * NBUF + b]], rows_v.at[b],
                gsem.at[b]).start()
        for b in range(NBUF):
            pltpu.make_async_copy(
                table_hbm.at[idx_v.at[g * NBUF + b]], rows_v.at[b],
                gsem.at[b]).wait()
            pltpu.make_async_copy(
                rows_v.at[b, :, pl.ds(0, EMBED_DIM)],
                out_hbm.at[wid, g * NBUF + b], ssem.at[b]).start()
        return carry

    lax.fori_loop(0, NOUTER, outer, 0)
    for b in range(NBUF):
        pltpu.make_async_copy(
            rows_v.at[b, :, pl.ds(0, EMBED_DIM)],
            out_hbm.at[wid, 0], ssem.at[b]).wait()


@functools.cache
def _make_sc_gather():
    return functools.partial(
        pl.kernel,
        mesh=plsc.VectorSubcoreMesh(
            core_axis_name="c", subcore_axis_name="s",
            num_cores=NC, num_subcores=NS),
        out_type=jax.ShapeDtypeStruct(
            (NW, NCHUNK, CHUNK, EMBED_DIM), jnp.float32),
        scratch_types=[
            pltpu.VMEM((NCHUNK, CHUNK), jnp.int32),
            pltpu.VMEM((NBUF, CHUNK, PAD_DIM), jnp.float32),
            pltpu.SemaphoreType.DMA((NBUF,)),
            pltpu.SemaphoreType.DMA((NBUF,)),
        ],
        compiler_params=pltpu.CompilerParams(use_tc_tiling_on_sc=False),
    )(_gather_body)


def kernel(input_ids, table):
    ids = input_ids.astype(jnp.int32).reshape(NW, NCHUNK, CHUNK)
    tpad = jnp.pad(table, ((0, 0), (0, PAD_DIM - EMBED_DIM)))
    out = _make_sc_gather()(tpad, ids)
    return out.reshape(BATCH, SEQ, EMBED_DIM)
